# baseline (speedup 1.0000x reference)
import jax
import jax.numpy as jnp
from jax import lax
from jax.experimental import pallas as pl
from jax.experimental.pallas import tpu as pltpu
from jax.experimental.pallas import tpu_sc as plsc

B = 16384
K = 32
NC, NS, L = 2, 16, 16
NW = NC * NS
BPW = B // NW
GRP = BPW // L
BLK = 128
NBUF = 4

_mesh = plsc.VectorSubcoreMesh(
    core_axis_name="c", subcore_axis_name="s", num_cores=NC, num_subcores=NS
)


def _body(uidx_hbm, iidx_hbm, utab_hbm, itab_hbm, out_hbm,
          uidx_v, iidx_v, ublk_v, iblk_v, upan_v, ipan_v, out_v, sems):
    wid = lax.axis_index("s") * NC + lax.axis_index("c")
    base = wid * BPW

    pltpu.sync_copy(uidx_hbm.at[pl.ds(base, BPW)], uidx_v)
    pltpu.sync_copy(iidx_hbm.at[pl.ds(base, BPW)], iidx_v)

    karange = lax.iota(jnp.int32, L)

    def sread(vref, e):
        v = vref[pl.ds((e >> 4) << 4, L)]
        m = karange == (e & (L - 1))
        return jnp.sum(jnp.where(m, v, 0))

    def fire(e, sl):
        ju = pl.multiple_of((sread(uidx_v, e) >> 7) * BLK, BLK)
        ji = pl.multiple_of((sread(iidx_v, e) >> 7) * BLK, BLK)
        pltpu.async_copy(
            utab_hbm.at[:, pl.ds(ju, BLK)], ublk_v.at[sl], sems.at[0, sl])
        pltpu.async_copy(
            itab_hbm.at[:, pl.ds(ji, BLK)], iblk_v.at[sl], sems.at[1, sl])

    def drain_u(sl):
        pltpu.make_async_copy(
            utab_hbm.at[:, pl.ds(0, BLK)], ublk_v.at[sl], sems.at[0, sl]
        ).wait()

    def drain_i(sl):
        pltpu.make_async_copy(
            itab_hbm.at[:, pl.ds(0, BLK)], iblk_v.at[sl], sems.at[1, sl]
        ).wait()

    def lane_splat(vref, e):
        return jnp.full((L,), sread(vref, e), jnp.int32)

    def extract_u(e, sl):
        cu = lane_splat(uidx_v, e) & (BLK - 1)
        pos = jnp.full((L,), e, jnp.int32)
        for half in range(2):
            ks = karange + half * L
            uv = plsc.load_gather(ublk_v.at[sl], [ks, cu])
            plsc.store_scatter(upan_v, [ks, pos], uv)

    def extract_i(e, sl):
        ci = lane_splat(iidx_v, e) & (BLK - 1)
        pos = jnp.full((L,), e, jnp.int32)
        for half in range(2):
            ks = karange + half * L
            iv = plsc.load_gather(iblk_v.at[sl], [ks, ci])
            plsc.store_scatter(ipan_v, [ks, pos], iv)

    for sl in range(NBUF):
        fire(sl, sl)

    def ring_body(g, carry):
        e0 = g * NBUF
        for j in range(NBUF):
            drain_u(j)
            extract_u(e0 + j, j)
            drain_i(j)
            extract_i(e0 + j, j)
            fire(e0 + NBUF + j, j)
        return carry

    lax.fori_loop(0, BPW // NBUF - 1, ring_body, 0)

    for j in range(NBUF):
        e = BPW - NBUF + j
        drain_u(j)
        extract_u(e, j)
        drain_i(j)
        extract_i(e, j)

    def g_body(g, carry):
        sl = pl.ds(g * L, L)
        acc = upan_v[0, sl] * ipan_v[0, sl]
        for k in range(1, K):
            acc = acc + upan_v[k, sl] * ipan_v[k, sl]
        out_v[sl] = acc
        return carry

    lax.fori_loop(0, GRP, g_body, 0)

    pltpu.sync_copy(out_v, out_hbm.at[pl.ds(base, BPW)])


_kern = pl.kernel(
    _body,
    out_type=jax.ShapeDtypeStruct((B,), jnp.float32),
    mesh=_mesh,
    scratch_types=[
        pltpu.VMEM((BPW,), jnp.int32),
        pltpu.VMEM((BPW,), jnp.int32),
        pltpu.VMEM((NBUF, K, BLK), jnp.float32),
        pltpu.VMEM((NBUF, K, BLK), jnp.float32),
        pltpu.VMEM((K, BPW), jnp.float32),
        pltpu.VMEM((K, BPW), jnp.float32),
        pltpu.VMEM((BPW,), jnp.float32),
        pltpu.SemaphoreType.DMA((2, NBUF)),
    ],
    compiler_params=pltpu.CompilerParams(
        needs_layout_passes=False, use_tc_tiling_on_sc=True),
)


@jax.jit
def kernel(user_input, item_input, user_embedding, item_embedding):
    out = _kern(user_input.reshape(B), item_input.reshape(B),
                user_embedding.T, item_embedding.T)
    return out.reshape(B, 1)


# speedup vs baseline: 1.0156x; 1.0156x over previous
import jax
import jax.numpy as jnp
from jax import lax
from jax.experimental import pallas as pl
from jax.experimental.pallas import tpu as pltpu
from jax.experimental.pallas import tpu_sc as plsc

B = 16384
K = 32
NC, NS, L = 2, 16, 16
NW = NC * NS
BPW = B // NW
GRP = BPW // L
BLK = 128
NBUF = 8

_mesh = plsc.VectorSubcoreMesh(
    core_axis_name="c", subcore_axis_name="s", num_cores=NC, num_subcores=NS
)


def _body(uidx_hbm, iidx_hbm, utab_hbm, itab_hbm, out_hbm,
          uidx_v, iidx_v, ublk_v, iblk_v, upan_v, ipan_v, out_v, sems):
    wid = lax.axis_index("s") * NC + lax.axis_index("c")
    base = wid * BPW

    pltpu.sync_copy(uidx_hbm.at[pl.ds(base, BPW)], uidx_v)
    pltpu.sync_copy(iidx_hbm.at[pl.ds(base, BPW)], iidx_v)

    karange = lax.iota(jnp.int32, L)

    def sread(vref, e):
        v = vref[pl.ds((e >> 4) << 4, L)]
        m = karange == (e & (L - 1))
        return jnp.sum(jnp.where(m, v, 0))

    def fire(e, sl):
        ju = pl.multiple_of((sread(uidx_v, e) >> 7) * BLK, BLK)
        ji = pl.multiple_of((sread(iidx_v, e) >> 7) * BLK, BLK)
        pltpu.async_copy(
            utab_hbm.at[:, pl.ds(ju, BLK)], ublk_v.at[sl], sems.at[0, sl])
        pltpu.async_copy(
            itab_hbm.at[:, pl.ds(ji, BLK)], iblk_v.at[sl], sems.at[1, sl])

    def drain_u(sl):
        pltpu.make_async_copy(
            utab_hbm.at[:, pl.ds(0, BLK)], ublk_v.at[sl], sems.at[0, sl]
        ).wait()

    def drain_i(sl):
        pltpu.make_async_copy(
            itab_hbm.at[:, pl.ds(0, BLK)], iblk_v.at[sl], sems.at[1, sl]
        ).wait()

    def lane_splat(vref, e):
        return jnp.full((L,), sread(vref, e), jnp.int32)

    def extract_u(e, sl):
        cu = lane_splat(uidx_v, e) & (BLK - 1)
        pos = jnp.full((L,), e, jnp.int32)
        for half in range(2):
            ks = karange + half * L
            uv = plsc.load_gather(ublk_v.at[sl], [ks, cu])
            plsc.store_scatter(upan_v, [ks, pos], uv)

    def extract_i(e, sl):
        ci = lane_splat(iidx_v, e) & (BLK - 1)
        pos = jnp.full((L,), e, jnp.int32)
        for half in range(2):
            ks = karange + half * L
            iv = plsc.load_gather(iblk_v.at[sl], [ks, ci])
            plsc.store_scatter(ipan_v, [ks, pos], iv)

    for sl in range(NBUF):
        fire(sl, sl)

    def ring_body(g, carry):
        e0 = g * NBUF
        for j in range(NBUF):
            drain_u(j)
            extract_u(e0 + j, j)
            drain_i(j)
            extract_i(e0 + j, j)
            fire(e0 + NBUF + j, j)
        return carry

    lax.fori_loop(0, BPW // NBUF - 1, ring_body, 0)

    for j in range(NBUF):
        e = BPW - NBUF + j
        drain_u(j)
        extract_u(e, j)
        drain_i(j)
        extract_i(e, j)

    def g_body(g, carry):
        sl = pl.ds(g * L, L)
        acc = upan_v[0, sl] * ipan_v[0, sl]
        for k in range(1, K):
            acc = acc + upan_v[k, sl] * ipan_v[k, sl]
        out_v[sl] = acc
        return carry

    lax.fori_loop(0, GRP, g_body, 0)

    pltpu.sync_copy(out_v, out_hbm.at[pl.ds(base, BPW)])


_kern = pl.kernel(
    _body,
    out_type=jax.ShapeDtypeStruct((B,), jnp.float32),
    mesh=_mesh,
    scratch_types=[
        pltpu.VMEM((BPW,), jnp.int32),
        pltpu.VMEM((BPW,), jnp.int32),
        pltpu.VMEM((NBUF, K, BLK), jnp.float32),
        pltpu.VMEM((NBUF, K, BLK), jnp.float32),
        pltpu.VMEM((K, BPW), jnp.float32),
        pltpu.VMEM((K, BPW), jnp.float32),
        pltpu.VMEM((BPW,), jnp.float32),
        pltpu.SemaphoreType.DMA((2, NBUF)),
    ],
    compiler_params=pltpu.CompilerParams(
        needs_layout_passes=False, use_tc_tiling_on_sc=True),
)


@jax.jit
def kernel(user_input, item_input, user_embedding, item_embedding):
    out = _kern(user_input.reshape(B), item_input.reshape(B),
                user_embedding.T, item_embedding.T)
    return out.reshape(B, 1)


# speedup vs baseline: 1.0189x; 1.0032x over previous
import jax
import jax.numpy as jnp
from jax import lax
from jax.experimental import pallas as pl
from jax.experimental.pallas import tpu as pltpu
from jax.experimental.pallas import tpu_sc as plsc

B = 16384
K = 32
NC, NS, L = 2, 16, 16
NW = NC * NS
BPW = B // NW
GRP = BPW // L
BLK = 128
NBUF = 8

_mesh = plsc.VectorSubcoreMesh(
    core_axis_name="c", subcore_axis_name="s", num_cores=NC, num_subcores=NS
)


def _body(uidx_hbm, iidx_hbm, utab_hbm, itab_hbm, out_hbm,
          uidx_v, iidx_v, ublk_v, iblk_v, upan_v, ipan_v, out_v, sems):
    wid = lax.axis_index("s") * NC + lax.axis_index("c")
    base = wid * BPW

    pltpu.sync_copy(uidx_hbm.at[pl.ds(base, BPW)], uidx_v)
    pltpu.sync_copy(iidx_hbm.at[pl.ds(base, BPW)], iidx_v)

    karange = lax.iota(jnp.int32, L)

    def sread(vref, e):
        v = vref[pl.ds((e >> 4) << 4, L)]
        m = karange == (e & (L - 1))
        return jnp.sum(jnp.where(m, v, 0))

    def fire(e, sl):
        ju = pl.multiple_of((sread(uidx_v, e) >> 7) * BLK, BLK)
        ji = pl.multiple_of((sread(iidx_v, e) >> 7) * BLK, BLK)
        pltpu.async_copy(
            utab_hbm.at[:, pl.ds(ju, BLK)], ublk_v.at[sl], sems.at[0, sl])
        pltpu.async_copy(
            itab_hbm.at[:, pl.ds(ji, BLK)], iblk_v.at[sl], sems.at[1, sl])

    def drain_u(sl):
        pltpu.make_async_copy(
            utab_hbm.at[:, pl.ds(0, BLK)], ublk_v.at[sl], sems.at[0, sl]
        ).wait()

    def drain_i(sl):
        pltpu.make_async_copy(
            itab_hbm.at[:, pl.ds(0, BLK)], iblk_v.at[sl], sems.at[1, sl]
        ).wait()

    def lane_splat(vref, e):
        return jnp.full((L,), sread(vref, e), jnp.int32)

    def extract_u(cu, pos, sl):
        for half in range(2):
            ks = karange + half * L
            uv = plsc.load_gather(ublk_v.at[sl], [ks, cu])
            plsc.store_scatter(upan_v, [ks, pos], uv)

    def extract_i(ci, pos, sl):
        for half in range(2):
            ks = karange + half * L
            iv = plsc.load_gather(iblk_v.at[sl], [ks, ci])
            plsc.store_scatter(ipan_v, [ks, pos], iv)

    for sl in range(NBUF):
        fire(sl, sl)

    def ring_body(g, carry):
        e0 = g * NBUF
        for j in range(NBUF):
            e = e0 + j
            cu = lane_splat(uidx_v, e) & (BLK - 1)
            ci = lane_splat(iidx_v, e) & (BLK - 1)
            pos = jnp.full((L,), e, jnp.int32)
            drain_u(j)
            extract_u(cu, pos, j)
            drain_i(j)
            extract_i(ci, pos, j)
            fire(e + NBUF, j)
        return carry

    lax.fori_loop(0, BPW // NBUF - 1, ring_body, 0)

    for j in range(NBUF):
        e = BPW - NBUF + j
        cu = lane_splat(uidx_v, e) & (BLK - 1)
        ci = lane_splat(iidx_v, e) & (BLK - 1)
        pos = jnp.full((L,), e, jnp.int32)
        drain_u(j)
        extract_u(cu, pos, j)
        drain_i(j)
        extract_i(ci, pos, j)

    def g_body(g, carry):
        sl = pl.ds(g * L, L)
        acc = upan_v[0, sl] * ipan_v[0, sl]
        for k in range(1, K):
            acc = acc + upan_v[k, sl] * ipan_v[k, sl]
        out_v[sl] = acc
        return carry

    lax.fori_loop(0, GRP, g_body, 0)

    pltpu.sync_copy(out_v, out_hbm.at[pl.ds(base, BPW)])


_kern = pl.kernel(
    _body,
    out_type=jax.ShapeDtypeStruct((B,), jnp.float32),
    mesh=_mesh,
    scratch_types=[
        pltpu.VMEM((BPW,), jnp.int32),
        pltpu.VMEM((BPW,), jnp.int32),
        pltpu.VMEM((NBUF, K, BLK), jnp.float32),
        pltpu.VMEM((NBUF, K, BLK), jnp.float32),
        pltpu.VMEM((K, BPW), jnp.float32),
        pltpu.VMEM((K, BPW), jnp.float32),
        pltpu.VMEM((BPW,), jnp.float32),
        pltpu.SemaphoreType.DMA((2, NBUF)),
    ],
    compiler_params=pltpu.CompilerParams(
        needs_layout_passes=False, use_tc_tiling_on_sc=True),
)


@jax.jit
def kernel(user_input, item_input, user_embedding, item_embedding):
    out = _kern(user_input.reshape(B), item_input.reshape(B),
                user_embedding.T, item_embedding.T)
    return out.reshape(B, 1)
